# Initial kernel scaffold; baseline (speedup 1.0000x reference)
import jax
import jax.numpy as jnp
from jax.experimental import pallas as pl


def kernel(user_id, item_id, user_table, item_table, adj_row, adj_col, adj_vals):
    raise NotImplementedError("write your pallas kernel here")


# speedup vs baseline: 2.9306x; 2.9306x over previous
import functools

import jax
import jax.numpy as jnp
from jax import lax
from jax.experimental import pallas as pl
from jax.experimental.pallas import tpu as pltpu
from jax.experimental.pallas import tpu_sc as plsc

N_USERS_K = 50000
N_ITEMS_K = 50000
NN = N_USERS_K + N_ITEMS_K
DD = 32
EE = 1600000
BB = 4096
ALPHA = 0.5
LAYERS = 3

NC = 2
NS = 16
LANES = 16

EPT = EE // NS
CH = 80
NCHUNK = EPT // CH
NN_PAD = 100096
ROWS_PT = NN_PAD // NS
ZR = 784

_mesh = plsc.VectorSubcoreMesh(core_axis_name="c", subcore_axis_name="s")


def _spmv_body(emb2, rows, cols, vals, out0, out1,
               colb, rowb, valb, gb, zb, acc, gsem):
    c = lax.axis_index("c")
    s = lax.axis_index("s")

    def zbody(i, carry):
        zb[i, :] = jnp.zeros((LANES,), jnp.float32)
        return carry
    lax.fori_loop(0, ZR, zbody, 0, unroll=8)
    for k in range(7):
        pltpu.sync_copy(zb, acc.at[pl.ds(s * ROWS_PT + k * ZR, ZR)])
    pltpu.sync_copy(zb.at[pl.ds(0, ROWS_PT - 7 * ZR)],
                    acc.at[pl.ds(s * ROWS_PT + 7 * ZR, ROWS_PT - 7 * ZR)])
    plsc.subcore_barrier()

    def chunk(j, carry):
        base = s * EPT + j * CH
        pltpu.sync_copy(rows.at[pl.ds(base, CH)], rowb.at[0])
        pltpu.sync_copy(cols.at[pl.ds(base, CH)], colb.at[0])
        pltpu.sync_copy(vals.at[pl.ds(base, CH)], valb)
        for i in range(CH // LANES):
            cv = colb[0, pl.ds(i * LANES, LANES)]
            colb[0, pl.ds(i * LANES, LANES)] = cv * 2 + c
        pltpu.async_copy(emb2.at[colb.at[0]], gb.at[0], gsem).wait()
        for i in range(CH // LANES):
            vv = valb[pl.ds(i * LANES, LANES)]
            for t in range(LANES):
                e = i * LANES + t
                gb[0, e, :] = gb[0, e, :] * vv[t]
        pltpu.sync_copy(gb.at[0], acc.at[rowb.at[0]], add=True)
        return carry

    lax.fori_loop(0, NCHUNK, chunk, 0)
    plsc.subcore_barrier()

    @pl.when(c == 0)
    def _():
        pltpu.sync_copy(acc.at[pl.ds(s * ROWS_PT, ROWS_PT)],
                        out0.at[pl.ds(s * ROWS_PT, ROWS_PT)])

    @pl.when(c == 1)
    def _():
        pltpu.sync_copy(acc.at[pl.ds(s * ROWS_PT, ROWS_PT)],
                        out1.at[pl.ds(s * ROWS_PT, ROWS_PT)])


_spmv = pl.kernel(
    _spmv_body,
    out_type=(jax.ShapeDtypeStruct((NN_PAD, 16), jnp.float32),
              jax.ShapeDtypeStruct((NN_PAD, 16), jnp.float32)),
    mesh=_mesh,
    scratch_types=[
        pltpu.VMEM((1, CH), jnp.int32),
        pltpu.VMEM((1, CH), jnp.int32),
        pltpu.VMEM((CH,), jnp.float32),
        pltpu.VMEM((1, CH, 16), jnp.float32),
        pltpu.VMEM((ZR, 16), jnp.float32),
        pltpu.VMEM_SHARED((NN_PAD, 16), jnp.float32),
        pltpu.SemaphoreType.DMA,
    ],
    compiler_params=pltpu.CompilerParams(use_tc_tiling_on_sc=False),
    name="lgcn_spmv_sc",
)


def _blend_body(final_layer, old_ref, n0_ref, n1_ref, acc_ref,
                emb_out_ref, acc_out_ref):
    old = old_ref[...]
    new = jnp.concatenate([n0_ref[...], n1_ref[...]], axis=-1)
    diff = old - new + 1e-6
    os_score = jnp.sqrt(jnp.sum(diff * diff, axis=1, keepdims=True))
    d_new = ALPHA * jnp.log1p(os_score)
    inv = 1.0 / (1.0 + d_new)
    emb = (old + d_new * new) * inv
    emb_out_ref[...] = emb
    acc = acc_ref[...] + emb
    if final_layer:
        acc = acc * 0.25
    acc_out_ref[...] = acc


def _make_blend(final_layer):
    blk = 1000
    grid = NN // blk
    return pl.pallas_call(
        functools.partial(_blend_body, final_layer),
        grid=(grid,),
        in_specs=[
            pl.BlockSpec((blk, DD), lambda i: (i, 0)),
            pl.BlockSpec((blk, 16), lambda i: (i, 0)),
            pl.BlockSpec((blk, 16), lambda i: (i, 0)),
            pl.BlockSpec((blk, DD), lambda i: (i, 0)),
        ],
        out_specs=[
            pl.BlockSpec((blk, DD), lambda i: (i, 0)),
            pl.BlockSpec((blk, DD), lambda i: (i, 0)),
        ],
        out_shape=[
            jax.ShapeDtypeStruct((NN, DD), jnp.float32),
            jax.ShapeDtypeStruct((NN, DD), jnp.float32),
        ],
        name="lgcn_blend_tc",
    )


_blend_mid = _make_blend(False)
_blend_last = _make_blend(True)


IDS_PT = BB // (NC * NS)


def _take_body(final_hbm, uid, iid, out_u, out_i, idxb, rbuf, gsem):
    c = lax.axis_index("c")
    s = lax.axis_index("s")
    w = s * NC + c
    base = w * IDS_PT

    pltpu.sync_copy(uid.at[pl.ds(base, IDS_PT)], idxb.at[0])
    pltpu.async_copy(final_hbm.at[idxb.at[0]], rbuf, gsem).wait()
    pltpu.sync_copy(rbuf, out_u.at[pl.ds(base, IDS_PT)])

    pltpu.sync_copy(iid.at[pl.ds(base, IDS_PT)], idxb.at[0])
    for i in range(IDS_PT // LANES):
        iv = idxb[0, pl.ds(i * LANES, LANES)]
        idxb[0, pl.ds(i * LANES, LANES)] = iv + N_USERS_K
    pltpu.async_copy(final_hbm.at[idxb.at[0]], rbuf, gsem).wait()
    pltpu.sync_copy(rbuf, out_i.at[pl.ds(base, IDS_PT)])


_take = pl.kernel(
    _take_body,
    out_type=(jax.ShapeDtypeStruct((BB, DD), jnp.float32),
              jax.ShapeDtypeStruct((BB, DD), jnp.float32)),
    mesh=_mesh,
    scratch_types=[
        pltpu.VMEM((1, IDS_PT), jnp.int32),
        pltpu.VMEM((IDS_PT, DD), jnp.float32),
        pltpu.SemaphoreType.DMA,
    ],
    compiler_params=pltpu.CompilerParams(use_tc_tiling_on_sc=False),
    name="lgcn_take_sc",
)


def kernel(user_id, item_id, user_table, item_table, adj_row, adj_col, adj_vals):
    ego = jnp.concatenate([user_table, item_table], axis=0)
    emb = ego
    acc = ego
    for layer in range(LAYERS):
        n0, n1 = _spmv(emb.reshape(2 * NN, 16), adj_row, adj_col, adj_vals)
        blend = _blend_last if layer == LAYERS - 1 else _blend_mid
        emb, acc = blend(emb, n0, n1, acc)
    u_embed, i_embed = _take(acc, user_id, item_id)
    return (u_embed, i_embed)


# speedup vs baseline: 9.6238x; 3.2839x over previous
import functools

import jax
import jax.numpy as jnp
from jax import lax
from jax.experimental import pallas as pl
from jax.experimental.pallas import tpu as pltpu
from jax.experimental.pallas import tpu_sc as plsc

N_USERS_K = 50000
N_ITEMS_K = 50000
NN = N_USERS_K + N_ITEMS_K
DD = 32
EE = 1600000
BB = 4096
ALPHA = 0.5
LAYERS = 3

NC = 2
NS = 16
LANES = 16

CH = 128
E_PAD = 1601536
NCHUNK = E_PAD // (NS * CH)
EPT = E_PAD // NS
NN_PAD = 100096
ROWS_PT = NN_PAD // NS
ZR = 784

_mesh = plsc.VectorSubcoreMesh(core_axis_name="c", subcore_axis_name="s")


def _spmv_body(emb2, rows, cols, vals, out0, out1,
               colb, rowb, valb, gb, zb, acc, sem_st, sem_g0, sem_g1):
    c = lax.axis_index("c")
    s = lax.axis_index("s")
    sem_g = (sem_g0, sem_g1)

    def zbody(i, carry):
        zb[i, :] = jnp.zeros((LANES,), jnp.float32)
        return carry
    lax.fori_loop(0, ZR, zbody, 0, unroll=8)
    for k in range(7):
        pltpu.sync_copy(zb, acc.at[pl.ds(s * ROWS_PT + k * ZR, ZR)])
    pltpu.sync_copy(zb.at[pl.ds(0, ROWS_PT - 7 * ZR)],
                    acc.at[pl.ds(s * ROWS_PT + 7 * ZR, ROWS_PT - 7 * ZR)])
    plsc.subcore_barrier()

    def issue_stage(j, b):
        base = s * EPT + j * CH
        pltpu.async_copy(rows.at[pl.ds(base, CH)], rowb.at[b], sem_st)
        pltpu.async_copy(cols.at[pl.ds(base, CH)], colb.at[b], sem_st)
        pltpu.async_copy(vals.at[pl.ds(base, CH)], valb.at[b], sem_st)

    def wait_stage(b):
        pltpu.make_async_copy(rows.at[pl.ds(0, CH)], rowb.at[b], sem_st).wait()
        pltpu.make_async_copy(cols.at[pl.ds(0, CH)], colb.at[b], sem_st).wait()
        pltpu.make_async_copy(vals.at[pl.ds(0, CH)], valb.at[b], sem_st).wait()

    def issue_gather(b):
        for i in range(CH // LANES):
            cv = colb[b, pl.ds(i * LANES, LANES)]
            colb[b, pl.ds(i * LANES, LANES)] = cv * 2 + c
        pltpu.async_copy(emb2.at[colb.at[b]], gb.at[b], sem_g[b])

    def wait_gather(b):
        pltpu.make_async_copy(emb2.at[colb.at[b]], gb.at[b], sem_g[b]).wait()

    def multiply(b):
        for i in range(CH // LANES):
            vv = valb[b, pl.ds(i * LANES, LANES)]
            for t in range(LANES):
                e = i * LANES + t
                gb[b, e, :] = gb[b, e, :] * vv[t]

    def scatter(b):
        pltpu.sync_copy(gb.at[b], acc.at[rowb.at[b]], add=True)

    issue_stage(0, 0)
    wait_stage(0)
    issue_gather(0)
    issue_stage(1, 1)

    def pair(g, carry):
        for b in range(2):
            j = 2 * g + b
            nb = 1 - b
            wait_stage(nb)
            issue_gather(nb)
            wait_gather(b)
            multiply(b)
            scatter(b)
            issue_stage(j + 2, b)
        return carry

    lax.fori_loop(0, (NCHUNK - 2) // 2, pair, 0)

    wait_stage(1)
    issue_gather(1)
    wait_gather(0)
    multiply(0)
    scatter(0)
    wait_gather(1)
    multiply(1)
    scatter(1)
    plsc.subcore_barrier()

    @pl.when(c == 0)
    def _():
        pltpu.sync_copy(acc.at[pl.ds(s * ROWS_PT, ROWS_PT)],
                        out0.at[pl.ds(s * ROWS_PT, ROWS_PT)])

    @pl.when(c == 1)
    def _():
        pltpu.sync_copy(acc.at[pl.ds(s * ROWS_PT, ROWS_PT)],
                        out1.at[pl.ds(s * ROWS_PT, ROWS_PT)])


_spmv = pl.kernel(
    _spmv_body,
    out_type=(jax.ShapeDtypeStruct((NN_PAD, 16), jnp.float32),
              jax.ShapeDtypeStruct((NN_PAD, 16), jnp.float32)),
    mesh=_mesh,
    scratch_types=[
        pltpu.VMEM((2, CH), jnp.int32),
        pltpu.VMEM((2, CH), jnp.int32),
        pltpu.VMEM((2, CH), jnp.float32),
        pltpu.VMEM((2, CH, 16), jnp.float32),
        pltpu.VMEM((ZR, 16), jnp.float32),
        pltpu.VMEM_SHARED((NN_PAD, 16), jnp.float32),
        pltpu.SemaphoreType.DMA,
        pltpu.SemaphoreType.DMA,
        pltpu.SemaphoreType.DMA,
    ],
    compiler_params=pltpu.CompilerParams(use_tc_tiling_on_sc=False),
    name="lgcn_spmv_sc",
)


def _blend_body(final_layer, old_ref, n0_ref, n1_ref, acc_ref,
                emb_out_ref, acc_out_ref):
    old = old_ref[...]
    new = jnp.concatenate([n0_ref[...], n1_ref[...]], axis=-1)
    diff = old - new + 1e-6
    os_score = jnp.sqrt(jnp.sum(diff * diff, axis=1, keepdims=True))
    d_new = ALPHA * jnp.log1p(os_score)
    inv = 1.0 / (1.0 + d_new)
    emb = (old + d_new * new) * inv
    emb_out_ref[...] = emb
    acc = acc_ref[...] + emb
    if final_layer:
        acc = acc * 0.25
    acc_out_ref[...] = acc


def _make_blend(final_layer):
    blk = 1000
    grid = NN // blk
    return pl.pallas_call(
        functools.partial(_blend_body, final_layer),
        grid=(grid,),
        in_specs=[
            pl.BlockSpec((blk, DD), lambda i: (i, 0)),
            pl.BlockSpec((blk, 16), lambda i: (i, 0)),
            pl.BlockSpec((blk, 16), lambda i: (i, 0)),
            pl.BlockSpec((blk, DD), lambda i: (i, 0)),
        ],
        out_specs=[
            pl.BlockSpec((blk, DD), lambda i: (i, 0)),
            pl.BlockSpec((blk, DD), lambda i: (i, 0)),
        ],
        out_shape=[
            jax.ShapeDtypeStruct((NN, DD), jnp.float32),
            jax.ShapeDtypeStruct((NN, DD), jnp.float32),
        ],
        name="lgcn_blend_tc",
    )


_blend_mid = _make_blend(False)
_blend_last = _make_blend(True)


IDS_PT = BB // (NC * NS)


def _take_body(final_hbm, uid, iid, out_u, out_i, idxb, rbuf, gsem):
    c = lax.axis_index("c")
    s = lax.axis_index("s")
    w = s * NC + c
    base = w * IDS_PT

    pltpu.sync_copy(uid.at[pl.ds(base, IDS_PT)], idxb.at[0])
    pltpu.async_copy(final_hbm.at[idxb.at[0]], rbuf, gsem).wait()
    pltpu.sync_copy(rbuf, out_u.at[pl.ds(base, IDS_PT)])

    pltpu.sync_copy(iid.at[pl.ds(base, IDS_PT)], idxb.at[0])
    for i in range(IDS_PT // LANES):
        iv = idxb[0, pl.ds(i * LANES, LANES)]
        idxb[0, pl.ds(i * LANES, LANES)] = iv + N_USERS_K
    pltpu.async_copy(final_hbm.at[idxb.at[0]], rbuf, gsem).wait()
    pltpu.sync_copy(rbuf, out_i.at[pl.ds(base, IDS_PT)])


_take = pl.kernel(
    _take_body,
    out_type=(jax.ShapeDtypeStruct((BB, DD), jnp.float32),
              jax.ShapeDtypeStruct((BB, DD), jnp.float32)),
    mesh=_mesh,
    scratch_types=[
        pltpu.VMEM((1, IDS_PT), jnp.int32),
        pltpu.VMEM((IDS_PT, DD), jnp.float32),
        pltpu.SemaphoreType.DMA,
    ],
    compiler_params=pltpu.CompilerParams(use_tc_tiling_on_sc=False),
    name="lgcn_take_sc",
)


def kernel(user_id, item_id, user_table, item_table, adj_row, adj_col, adj_vals):
    ego = jnp.concatenate([user_table, item_table], axis=0)
    pad = E_PAD - EE
    rows_p = jnp.concatenate([adj_row, jnp.zeros((pad,), jnp.int32)])
    cols_p = jnp.concatenate([adj_col, jnp.zeros((pad,), jnp.int32)])
    vals_p = jnp.concatenate([adj_vals, jnp.zeros((pad,), jnp.float32)])
    emb = ego
    acc = ego
    for layer in range(LAYERS):
        n0, n1 = _spmv(emb.reshape(2 * NN, 16), rows_p, cols_p, vals_p)
        blend = _blend_last if layer == LAYERS - 1 else _blend_mid
        emb, acc = blend(emb, n0, n1, acc)
    u_embed, i_embed = _take(acc, user_id, item_id)
    return (u_embed, i_embed)
